# baseline (speedup 1.0000x reference)
import functools

import jax
import jax.numpy as jnp
from jax import lax
from jax.experimental import pallas as pl
from jax.experimental.pallas import tpu as pltpu
from jax.experimental.pallas import tpu_sc as plsc

EMBED = 64

_info = plsc.get_sparse_core_info()
_NC = _info.num_cores
_NS = _info.num_subcores
_NW = _NC * _NS


def _emb_body(bpw, hist, table2, idx3, out3, idx_v, tid_v,
              tb0, tb1, tb2, tb3, ob0, ob1, ob2, ob3,
              gs0, gs1, gs2, gs3, os0, os1, os2, os3):
    wid = lax.axis_index("s") * _NC + lax.axis_index("c")
    tbufs = (tb0, tb1, tb2, tb3)
    obufs = (ob0, ob1, ob2, ob3)
    gsems = (gs0, gs1, gs2, gs3)
    osems = (os0, os1, os2, os3)

    pltpu.sync_copy(idx3.at[wid], idx_v)

    wins = []
    covered = 0
    s = 0
    while covered < hist:
        base = min(s, hist - 16)
        hi = min(hist, base + 16)
        wins.append((base, covered, hi))
        covered = hi
        s = base + 16

    for j in range(bpw):
        for base, _, _ in wins:
            tid_v[j, pl.ds(base, 16)] = (
                lax.shift_right_logical(idx_v[j, pl.ds(base, 16)], 1))

    def issue_gather(j, pp):
        pltpu.async_copy(table2.at[tid_v.at[j]], tbufs[pp], gsems[pp])

    def wait_gather(pp):
        pltpu.make_async_copy(table2.at[tid_v.at[0]], tbufs[pp],
                              gsems[pp]).wait()

    def issue_out(j, pp):
        pltpu.async_copy(obufs[pp], out3.at[wid * bpw + j], osems[pp])

    def wait_out(pp):
        pltpu.make_async_copy(obufs[pp], out3.at[0], osems[pp]).wait()

    def select(j, pp):
        for base, lo, hi in wins:
            v = idx_v[j, pl.ds(base, 16)]
            for lane in range(lo - base, hi - base):
                off = (v[lane] & 1) << 6
                r = base + lane
                for k in range(EMBED // 16):
                    obufs[pp][r, pl.ds(k * 16, 16)] = (
                        tbufs[pp][r, pl.ds(off + k * 16, 16)])

    for g in range(4):
        issue_gather(g, g)

    def outer(j2, carry):
        for pp in range(4):
            j = j2 * 4 + pp
            wait_gather(pp)
            qq = (pp + 2) % 4

            @pl.when((j >= 2) & (j + 2 < bpw))
            def _():
                wait_out(qq)
                issue_gather(j + 2, qq)

            select(j, pp)
            issue_out(j, pp)
        return carry

    lax.fori_loop(0, bpw // 4, outer, 0)

    for pp in range(4):
        wait_out(pp)


def kernel(indices, table):
    batch, hist = indices.shape
    vocab = table.shape[0]
    assert batch % _NW == 0 and vocab % 2 == 0
    bpw = batch // _NW
    assert bpw % 4 == 0 and bpw >= 8
    table2 = table.reshape(vocab // 2, 2 * EMBED)
    idx3 = indices.reshape(_NW, bpw, hist)

    k = pl.kernel(
        functools.partial(_emb_body, bpw, hist),
        out_type=jax.ShapeDtypeStruct((batch, hist, EMBED), jnp.float32),
        mesh=plsc.VectorSubcoreMesh(core_axis_name="c", subcore_axis_name="s"),
        scratch_types=(
            [pltpu.VMEM((bpw, hist), jnp.int32),
             pltpu.VMEM((bpw, hist), jnp.int32)]
            + [pltpu.VMEM((hist, 2 * EMBED), jnp.float32) for _ in range(4)]
            + [pltpu.VMEM((hist, EMBED), jnp.float32) for _ in range(4)]
            + [pltpu.SemaphoreType.DMA for _ in range(8)]
        ),
    )
    return k(table2, idx3)


# speedup vs baseline: 2.0506x; 2.0506x over previous
import functools

import jax
import jax.numpy as jnp
from jax import lax
from jax.experimental import pallas as pl
from jax.experimental.pallas import tpu as pltpu
from jax.experimental.pallas import tpu_sc as plsc

EMBED = 64

_info = plsc.get_sparse_core_info()
_NC = _info.num_cores
_NS = _info.num_subcores
_NW = _NC * _NS


def _emb_body(bpw, hist, table_hbm, idx3, out3, idx_v, ob0, ob1, ob2, ob3,
              gs0, gs1, gs2, gs3, os0, os1, os2, os3):
    wid = lax.axis_index("s") * _NC + lax.axis_index("c")
    obufs = (ob0, ob1, ob2, ob3)
    gsems = (gs0, gs1, gs2, gs3)
    osems = (os0, os1, os2, os3)

    pltpu.sync_copy(idx3.at[wid], idx_v)

    def issue_gathers(j, pp):
        s = 0
        while s < hist:
            base = min(s, hist - 16)
            v = idx_v[j, pl.ds(base, 16)]
            for lane in range(s - base, min(hist, base + 16) - base):
                e = v[lane]
                pltpu.async_copy(table_hbm.at[e >> 3, e & 7],
                                 obufs[pp].at[base + lane], gsems[pp])
            s = base + 16

    def wait_gathers(pp):
        pltpu.make_async_copy(out3.at[0], obufs[pp], gsems[pp]).wait()

    def issue_out(j, pp):
        pltpu.async_copy(obufs[pp], out3.at[wid * bpw + j], osems[pp])

    def wait_out(pp):
        pltpu.make_async_copy(obufs[pp], out3.at[0], osems[pp]).wait()

    for g in range(4):
        issue_gathers(g, g)

    def outer(j2, carry):
        for pp in range(4):
            j = j2 * 4 + pp
            wait_gathers(pp)
            issue_out(j, pp)
            qq = (pp + 2) % 4

            @pl.when((j >= 2) & (j + 2 < bpw))
            def _():
                wait_out(qq)
                issue_gathers(j + 2, qq)
        return carry

    lax.fori_loop(0, bpw // 4, outer, 0)

    for pp in range(4):
        wait_out(pp)


def kernel(indices, table):
    batch, hist = indices.shape
    vocab = table.shape[0]
    assert batch % _NW == 0 and vocab % 8 == 0
    bpw = batch // _NW
    assert bpw % 4 == 0 and bpw >= 8
    table3 = table.reshape(vocab // 8, 8, EMBED)
    idx3 = indices.reshape(_NW, bpw, hist)

    k = pl.kernel(
        functools.partial(_emb_body, bpw, hist),
        out_type=jax.ShapeDtypeStruct((batch, hist, EMBED), jnp.float32),
        mesh=plsc.VectorSubcoreMesh(core_axis_name="c", subcore_axis_name="s"),
        scratch_types=(
            [pltpu.VMEM((bpw, hist), jnp.int32)]
            + [pltpu.VMEM((hist, EMBED), jnp.float32) for _ in range(4)]
            + [pltpu.SemaphoreType.DMA for _ in range(8)]
        ),
    )
    return k(table3, idx3)
